# baseline (speedup 1.0000x reference)
import functools

import jax
import jax.numpy as jnp
from jax import lax
from jax.experimental import pallas as pl
from jax.experimental.pallas import tpu as pltpu
from jax.experimental.pallas import tpu_sc as plsc

D = 32
NC, NS = 2, 16
NW = NC * NS
U = 4
TB = 128
W = U * TB


@functools.lru_cache(maxsize=None)
def _make_gather(B: int, H: int):
    assert B == NW * U * TB and H % 2 == 0
    mesh = plsc.VectorSubcoreMesh(core_axis_name="c", subcore_axis_name="s")

    @functools.partial(
        pl.kernel,
        out_type=jax.ShapeDtypeStruct((H, D // 8, B // TB, 8, TB),
                                      jnp.float32),
        mesh=mesh,
        scratch_types=[
            pltpu.VMEM((2, W), jnp.int32),
            pltpu.VMEM((2, W, D), jnp.float32),
            pltpu.VMEM((2, U, D, TB + 1), jnp.float32),
            pltpu.SemaphoreType.DMA,
            pltpu.SemaphoreType.DMA,
        ],
        compiler_params=pltpu.CompilerParams(use_tc_tiling_on_sc=False,
                                             needs_layout_passes=False),
    )
    def body(idxt_hbm, tab_hbm, out_hbm, idx_v, gath_v, trans_v, gsem, osem):
        wid = lax.axis_index("s") * NC + lax.axis_index("c")
        lane = lax.iota(jnp.int32, 16)
        d_lo, d_hi = lane, lane + 16

        def fire_gathers(h, buf):
            pltpu.sync_copy(idxt_hbm.at[h, pl.ds(wid * W, W)], idx_v.at[buf])
            for u in range(U):
                pltpu.async_copy(
                    tab_hbm.at[idx_v.at[buf, pl.ds(u * TB, TB)]],
                    gath_v.at[buf, pl.ds(u * TB, TB)],
                    gsem,
                )

        def drain_gathers(buf):
            for u in range(U):
                pltpu.make_async_copy(
                    tab_hbm.at[idx_v.at[buf, pl.ds(u * TB, TB)]],
                    gath_v.at[buf, pl.ds(u * TB, TB)],
                    gsem,
                ).wait()

        def transpose(buf):
            @plsc.parallel_loop(0, TB, unroll=8)
            def _tok(t):
                t_splat = jnp.full((16,), t, jnp.int32)
                for u in range(U):
                    row = u * TB + t
                    v0 = gath_v[buf, row, pl.ds(0, 16)]
                    v1 = gath_v[buf, row, pl.ds(16, 16)]
                    plsc.store_scatter(trans_v.at[buf, u], [d_lo, t_splat], v0)
                    plsc.store_scatter(trans_v.at[buf, u], [d_hi, t_splat], v1)

        def fire_stores(h, buf):
            for u in range(U):
                for dt in range(D // 8):
                    pltpu.async_copy(
                        trans_v.at[buf, u, pl.ds(dt * 8, 8), pl.ds(0, TB)],
                        out_hbm.at[h, dt, wid * U + u],
                        osem,
                    )

        def wait_stores(h, buf):
            for u in range(U):
                for dt in range(D // 8):
                    pltpu.make_async_copy(
                        trans_v.at[buf, u, pl.ds(dt * 8, 8), pl.ds(0, TB)],
                        out_hbm.at[h, dt, wid * U + u],
                        osem,
                    ).wait()

        fire_gathers(0, 0)
        fire_gathers(1, 1)
        for g in range(2):
            drain_gathers(g)
            transpose(g)
            fire_stores(g, g)
            fire_gathers(g + 2, g)

        @pl.loop(2, H - 2, step=2)
        def _pair(g0):
            for buf in range(2):
                g = g0 + buf
                drain_gathers(buf)
                wait_stores(g - 2, buf)
                transpose(buf)
                fire_stores(g, buf)
                fire_gathers(g + 2, buf)

        for i in range(2):
            g = H - 2 + i
            drain_gathers(i)
            wait_stores(g - 2, i)
            transpose(i)
            fire_stores(g, i)
        for i in range(2):
            wait_stores(H - 2 + i, i)

    return body


CH = 512


@functools.lru_cache(maxsize=None)
def _make_etrans(V: int):
    PW = 61 * CH
    NJ = 61
    TAIL0 = 31 * PW + NJ * CH
    assert TAIL0 == 32 * PW
    TAILN = V - 32 * PW
    mesh = plsc.VectorSubcoreMesh(core_axis_name="c", subcore_axis_name="s")

    @functools.partial(
        pl.kernel,
        out_type=jax.ShapeDtypeStruct((V, D), jnp.float32),
        mesh=mesh,
        scratch_types=[
            pltpu.VMEM((2, D, CH), jnp.float32),
            pltpu.VMEM((2, CH, D + 1), jnp.float32),
            pltpu.SemaphoreType.DMA,
            pltpu.SemaphoreType.DMA,
        ],
        compiler_params=pltpu.CompilerParams(use_tc_tiling_on_sc=False,
                                             needs_layout_passes=False),
    )
    def body(et_hbm, elin_hbm, in_v, tout_v, isem, osem):
        wid = lax.axis_index("s") * NC + lax.axis_index("c")
        lane = lax.iota(jnp.int32, 16)
        c0 = wid * PW

        def fire_in(j, buf):
            pltpu.async_copy(et_hbm.at[:, pl.ds(c0 + j * CH, CH)],
                             in_v.at[buf], isem)

        def wait_in(j, buf):
            pltpu.make_async_copy(et_hbm.at[:, pl.ds(c0 + j * CH, CH)],
                                  in_v.at[buf], isem).wait()

        def transpose(buf, ncols):
            @plsc.parallel_loop(0, ncols // 16, unroll=4)
            def _grp(ig):
                i_idx = lane + ig * 16
                for d in range(D):
                    v = in_v[buf, d, pl.ds(ig * 16, 16)]
                    plsc.store_scatter(
                        tout_v.at[buf], [i_idx, jnp.full((16,), d, jnp.int32)],
                        v)

        def fire_out(j, buf):
            pltpu.async_copy(tout_v.at[buf, :, pl.ds(0, D)],
                             elin_hbm.at[pl.ds(c0 + j * CH, CH)], osem)

        def wait_out(j, buf):
            pltpu.make_async_copy(tout_v.at[buf, :, pl.ds(0, D)],
                                  elin_hbm.at[pl.ds(c0 + j * CH, CH)],
                                  osem).wait()

        def step(j, buf, first, fire_next):
            wait_in(j, buf)
            if not first:
                wait_out(j - 2, buf)
            transpose(buf, CH)
            fire_out(j, buf)
            if fire_next:
                fire_in(j + 2, buf)

        fire_in(0, 0)
        fire_in(1, 1)
        step(0, 0, True, True)
        step(1, 1, True, True)

        @pl.loop(2, NJ - 3, step=2)
        def _pair(j0):
            for buf in range(2):
                step(j0 + buf, buf, False, True)

        step(NJ - 3, 0, False, True)
        step(NJ - 2, 1, False, False)
        step(NJ - 1, 0, False, False)
        wait_out(NJ - 2, 1)
        wait_out(NJ - 1, 0)

        @pl.when(wid == NW - 1)
        def _tail():
            t0 = 32 * PW - c0
            pltpu.sync_copy(et_hbm.at[:, pl.ds(c0 + t0, CH)], in_v.at[0])
            transpose(0, CH)
            pltpu.sync_copy(tout_v.at[0, :, pl.ds(0, D)],
                            elin_hbm.at[pl.ds(c0 + t0, CH)])
            t1 = t0 + CH
            n1 = TAILN - CH
            pltpu.sync_copy(et_hbm.at[:, pl.ds(c0 + t1, n1)],
                            in_v.at[0, :, pl.ds(0, n1)])

            @plsc.parallel_loop(0, n1 // 16, unroll=4)
            def _grp(ig):
                i_idx = lane + ig * 16
                for d in range(D):
                    v = in_v[0, d, pl.ds(ig * 16, 16)]
                    plsc.store_scatter(
                        tout_v.at[0], [i_idx, jnp.full((16,), d, jnp.int32)],
                        v)

            pltpu.sync_copy(tout_v.at[0, pl.ds(0, n1), pl.ds(0, D)],
                            elin_hbm.at[pl.ds(c0 + t1, n1)])

    return body


def kernel(x, E):
    B, H = x.shape
    V = E.shape[0]
    elin = _make_etrans(V)(E.T)
    out5 = _make_gather(B, H)(x.astype(jnp.int32).T, elin)
    return out5.transpose(2, 4, 0, 1, 3).reshape(B, H, D)


# speedup vs baseline: 3.4563x; 3.4563x over previous
import functools

import jax
import jax.numpy as jnp
from jax import lax
from jax.experimental import pallas as pl
from jax.experimental.pallas import tpu as pltpu
from jax.experimental.pallas import tpu_sc as plsc

D = 32
NC, NS = 2, 16
NW = NC * NS
U = 4
TB = 128
W = U * TB


@functools.lru_cache(maxsize=None)
def _make_gather(B: int, H: int):
    assert B == NW * U * TB and H % 2 == 0
    mesh = plsc.VectorSubcoreMesh(core_axis_name="c", subcore_axis_name="s")

    @functools.partial(
        pl.kernel,
        out_type=jax.ShapeDtypeStruct((H, D // 8, B // TB, 8, TB),
                                      jnp.float32),
        mesh=mesh,
        scratch_types=[
            pltpu.VMEM((2, W), jnp.int32),
            pltpu.VMEM((2, W, D), jnp.float32),
            pltpu.VMEM((2, U, D, TB + 1), jnp.float32),
            pltpu.SemaphoreType.DMA,
            pltpu.SemaphoreType.DMA,
        ],
        compiler_params=pltpu.CompilerParams(use_tc_tiling_on_sc=False,
                                             needs_layout_passes=False),
    )
    def body(idxt_hbm, tab_hbm, out_hbm, idx_v, gath_v, trans_v, gsem, osem):
        wid = lax.axis_index("s") * NC + lax.axis_index("c")
        lane = lax.iota(jnp.int32, 16)
        d_lo, d_hi = lane, lane + 16

        def fire_gathers(h, buf):
            pltpu.sync_copy(idxt_hbm.at[h, pl.ds(wid * W, W)], idx_v.at[buf])
            for u in range(U):
                pltpu.async_copy(
                    tab_hbm.at[idx_v.at[buf, pl.ds(u * TB, TB)]],
                    gath_v.at[buf, pl.ds(u * TB, TB)],
                    gsem,
                )

        def drain_gathers(buf):
            for u in range(U):
                pltpu.make_async_copy(
                    tab_hbm.at[idx_v.at[buf, pl.ds(u * TB, TB)]],
                    gath_v.at[buf, pl.ds(u * TB, TB)],
                    gsem,
                ).wait()

        def transpose(buf):
            @plsc.parallel_loop(0, TB, unroll=8)
            def _tok(t):
                t_splat = jnp.full((16,), t, jnp.int32)
                for u in range(U):
                    row = u * TB + t
                    v0 = gath_v[buf, row, pl.ds(0, 16)]
                    v1 = gath_v[buf, row, pl.ds(16, 16)]
                    plsc.store_scatter(trans_v.at[buf, u], [d_lo, t_splat], v0)
                    plsc.store_scatter(trans_v.at[buf, u], [d_hi, t_splat], v1)

        def fire_stores(h, buf):
            for u in range(U):
                for dt in range(D // 8):
                    pltpu.async_copy(
                        trans_v.at[buf, u, pl.ds(dt * 8, 8), pl.ds(0, TB)],
                        out_hbm.at[h, dt, wid * U + u],
                        osem,
                    )

        def wait_stores(h, buf):
            for u in range(U):
                for dt in range(D // 8):
                    pltpu.make_async_copy(
                        trans_v.at[buf, u, pl.ds(dt * 8, 8), pl.ds(0, TB)],
                        out_hbm.at[h, dt, wid * U + u],
                        osem,
                    ).wait()

        fire_gathers(0, 0)
        fire_gathers(1, 1)
        for g in range(2):
            drain_gathers(g)
            transpose(g)
            fire_stores(g, g)
            fire_gathers(g + 2, g)

        @pl.loop(2, H - 2, step=2)
        def _pair(g0):
            for buf in range(2):
                g = g0 + buf
                drain_gathers(buf)
                wait_stores(g - 2, buf)
                transpose(buf)
                fire_stores(g, buf)
                fire_gathers(g + 2, buf)

        for i in range(2):
            g = H - 2 + i
            drain_gathers(i)
            wait_stores(g - 2, i)
            transpose(i)
            fire_stores(g, i)
        for i in range(2):
            wait_stores(H - 2 + i, i)

    return body


def kernel(x, E):
    B, H = x.shape
    out5 = _make_gather(B, H)(x.astype(jnp.int32).T, E)
    return out5.transpose(2, 4, 0, 1, 3).reshape(B, H, D)
